# baseline (speedup 1.0000x reference)
import functools
import math

import jax
import jax.numpy as jnp
from jax import lax
from jax.experimental import pallas as pl
from jax.experimental.pallas import tpu as pltpu
from jax.experimental.pallas import tpu_sc as plsc

N = 4096
V = 32000
PAD = 0
SMOOTH = 0.1
CONF = 1.0 - SMOOTH
EPS = SMOOTH / (V - 2)
ROW_CONST = CONF * math.log(CONF) + (V - 2) * EPS * math.log(EPS)

NC = 2
NS = 16
L = 16
NW = NC * NS
BPW = N // NW

ROWS_SC = 2048
ROWS_TC = N - ROWS_SC
RPW = ROWS_SC // NW

UNROLL = 20
SLICES = V // L
ITERS = SLICES // UNROLL

BR = 128
NRB = ROWS_TC // BR


def _sc_part(x1, x2d, y):
    mesh = plsc.VectorSubcoreMesh(core_axis_name="c", subcore_axis_name="s")

    @functools.partial(
        pl.kernel,
        mesh=mesh,
        out_type=[
            jax.ShapeDtypeStruct((NW, L), jnp.float32),
            jax.ShapeDtypeStruct((NW, L), jnp.float32),
            jax.ShapeDtypeStruct((NW, RPW * L), jnp.float32),
        ],
        scratch_types=[
            pltpu.VMEM((BPW,), jnp.int32),
            pltpu.VMEM((BPW,), jnp.int32),
            pltpu.VMEM((BPW,), jnp.float32),
            pltpu.VMEM((RPW * L,), jnp.float32),
            pltpu.VMEM((2, V), jnp.float32),
            pltpu.VMEM((L,), jnp.float32),
            pltpu.VMEM((L,), jnp.float32),
            pltpu.SemaphoreType.DMA,
            pltpu.SemaphoreType.DMA,
            pltpu.SemaphoreType.DMA,
        ],
    )
    def k(x1_hbm, x_hbm, y_hbm, g_hbm, n_hbm, b_hbm,
          y_v, idx_v, vals_v, bvec_v, rowbuf, g_v, n_v,
          sem, sem0, sem1):
        wid = lax.axis_index("s") * NC + lax.axis_index("c")
        iv = lax.iota(jnp.int32, L)

        base = wid * BPW
        pltpu.sync_copy(y_hbm.at[pl.ds(base, BPW)], y_v)
        for j in range(BPW // L):
            yv = y_v[pl.ds(j * L, L)]
            row = base + j * L + iv
            idx_v[pl.ds(j * L, L)] = row * V + yv
        gather_cp = pltpu.async_copy(x1_hbm.at[idx_v], vals_v, sem)

        row0 = ROWS_TC + wid * RPW
        sems = (sem0, sem1)
        cps = [pltpu.async_copy(x_hbm.at[row0], rowbuf.at[0], sem0), None]

        def reduce_body(i, accs):
            a0, a1, a2, a3 = accs
            off = i * (UNROLL * L)
            for s in range(UNROLL):
                sl = buf[pl.ds(off + s * L, L)]
                if s % 4 == 0:
                    a0 = a0 + sl
                elif s % 4 == 1:
                    a1 = a1 + sl
                elif s % 4 == 2:
                    a2 = a2 + sl
                else:
                    a3 = a3 + sl
            return a0, a1, a2, a3

        zero = jnp.zeros((L,), jnp.float32)
        for r in range(RPW):
            if r + 1 < RPW:
                cps[(r + 1) % 2] = pltpu.async_copy(
                    x_hbm.at[row0 + r + 1], rowbuf.at[(r + 1) % 2],
                    sems[(r + 1) % 2])
            cps[r % 2].wait()
            buf = rowbuf.at[r % 2]
            a0, a1, a2, a3 = lax.fori_loop(
                0, ITERS, reduce_body, (zero, zero, zero, zero))
            tot = (a0 + a1) + (a2 + a3)
            c0vec = jnp.where(iv == 0, buf[pl.ds(0, L)], 0.0)
            bvec_v[pl.ds(r * L, L)] = c0vec - tot

        gather_cp.wait()
        acc = jnp.zeros((L,), jnp.float32)
        cnt = jnp.zeros((L,), jnp.float32)
        for j in range(BPW // L):
            yv = y_v[pl.ds(j * L, L)]
            g = vals_v[pl.ds(j * L, L)]
            m = yv != PAD
            acc = acc + jnp.where(m, g, 0.0)
            cnt = cnt + jnp.where(m, 1.0, 0.0)
        g_v[...] = acc
        n_v[...] = cnt

        pltpu.sync_copy(g_v, g_hbm.at[wid])
        pltpu.sync_copy(n_v, n_hbm.at[wid])
        pltpu.sync_copy(bvec_v, b_hbm.at[wid])

    return k(x1, x2d, y)


def _tc_body(y_ref, x_ref, out_ref):
    blk = x_ref[...]
    s = jnp.sum(blk, axis=1, keepdims=True)
    mask = y_ref[0] != PAD
    val = jnp.sum(jnp.where(mask, blk[:, 0:1] - s, 0.0))
    r = pl.program_id(0)

    @pl.when(r == 0)
    def _():
        out_ref[0, 0] = val

    @pl.when(r != 0)
    def _():
        out_ref[0, 0] = out_ref[0, 0] + val


def _tc_masked_colsum(x, y3):
    return pl.pallas_call(
        _tc_body,
        grid=(NRB,),
        in_specs=[
            pl.BlockSpec((1, BR, 1), lambda r: (r, 0, 0)),
            pl.BlockSpec((BR, V), lambda r: (r, 0)),
        ],
        out_specs=pl.BlockSpec(memory_space=pltpu.SMEM),
        out_shape=jax.ShapeDtypeStruct((1, 1), jnp.float32),
    )(y3, x)


def _tc2_body(b_ref, y_ref, out_ref):
    mask = y_ref[...] != PAD
    out_ref[0, 0] = jnp.sum(jnp.where(mask, b_ref[...], 0.0))


def _tc2_masked_reduce(b_rows, y2d):
    return pl.pallas_call(
        _tc2_body,
        grid=(1,),
        in_specs=[
            pl.BlockSpec((ROWS_SC, L), lambda i: (0, 0)),
            pl.BlockSpec((ROWS_SC, 1), lambda i: (1, 0)),
        ],
        out_specs=pl.BlockSpec(memory_space=pltpu.SMEM),
        out_shape=jax.ShapeDtypeStruct((1, 1), jnp.float32),
    )(b_rows, y2d)


def kernel(x, y):
    g_part, n_part, b_part = _sc_part(x.reshape(N * V), x, y)
    b_tc = _tc_masked_colsum(x, y.reshape(N // BR, BR, 1))
    b_sc = _tc2_masked_reduce(b_part.reshape(ROWS_SC, L), y.reshape(N, 1))
    a = jnp.sum(g_part)
    cnt = jnp.sum(n_part)
    b = b_tc[0, 0] + b_sc[0, 0]
    return (cnt * jnp.float32(ROW_CONST)
            - jnp.float32(CONF - EPS) * a
            + jnp.float32(EPS) * b)


# speedup vs baseline: 2.5126x; 2.5126x over previous
import functools
import math

import jax
import jax.numpy as jnp
from jax import lax
from jax.experimental import pallas as pl
from jax.experimental.pallas import tpu as pltpu
from jax.experimental.pallas import tpu_sc as plsc

N = 4096
V = 32000
PAD = 0
SMOOTH = 0.1
CONF = 1.0 - SMOOTH
EPS = SMOOTH / (V - 2)
ROW_CONST = CONF * math.log(CONF) + (V - 2) * EPS * math.log(EPS)

NC = 2
NS = 16
L = 16
NW = NC * NS

ROWS_SC = 1024
ROWS_TC = N - ROWS_SC
RPW = ROWS_SC // NW
IDX_BATCH = 128
SPLAT_BATCHES = RPW * L // IDX_BATCH

UNROLL = 25
SLICES = V // L
ITERS = SLICES // UNROLL

BR = 128
NRB = ROWS_TC // BR


def _sc_part(x, y):
    mesh = plsc.VectorSubcoreMesh(core_axis_name="c", subcore_axis_name="s")

    @functools.partial(
        pl.kernel,
        mesh=mesh,
        out_type=[
            jax.ShapeDtypeStruct((NW, L), jnp.float32),
            jax.ShapeDtypeStruct((NW, L), jnp.float32),
            jax.ShapeDtypeStruct((NW, L), jnp.float32),
        ],
        scratch_types=[
            pltpu.VMEM((RPW * L,), jnp.int32),
            pltpu.VMEM((RPW * L,), jnp.int32),
            pltpu.VMEM((2, V), jnp.float32),
            pltpu.VMEM((L,), jnp.float32),
            pltpu.VMEM((L,), jnp.float32),
            pltpu.VMEM((L,), jnp.float32),
            pltpu.SemaphoreType.DMA,
            pltpu.SemaphoreType.DMA,
            pltpu.SemaphoreType.DMA,
        ],
    )
    def k(x_hbm, y_hbm, g_hbm, b_hbm, k_hbm,
          idxs_v, yspl_v, rowbuf, g_v, b_v, k_v,
          sem, sem0, sem1):
        wid = lax.axis_index("s") * NC + lax.axis_index("c")
        iv = lax.iota(jnp.int32, L)
        row0 = ROWS_TC + wid * RPW

        zi = jnp.zeros((L,), jnp.int32)
        for j in range(RPW):
            idxs_v[pl.ds(j * L, L)] = zi + (row0 + j)
        for j in range(SPLAT_BATCHES):
            pltpu.async_copy(
                y_hbm.at[idxs_v.at[pl.ds(j * IDX_BATCH, IDX_BATCH)]],
                yspl_v.at[pl.ds(j * IDX_BATCH, IDX_BATCH)], sem).wait()

        sems = (sem0, sem1)
        cps = [pltpu.async_copy(x_hbm.at[row0], rowbuf.at[0], sem0), None]

        gacc = jnp.zeros((L,), jnp.float32)
        bacc = jnp.zeros((L,), jnp.float32)
        kacc = jnp.zeros((L,), jnp.float32)
        zero = jnp.zeros((L,), jnp.float32)
        fone = jnp.ones((L,), jnp.float32)

        for r in range(RPW):
            if r + 1 < RPW:
                cps[(r + 1) % 2] = pltpu.async_copy(
                    x_hbm.at[row0 + r + 1], rowbuf.at[(r + 1) % 2],
                    sems[(r + 1) % 2])
            cps[r % 2].wait()
            buf = rowbuf.at[r % 2]
            yspl = yspl_v[pl.ds(r * L, L)]

            def reduce_body(i, accs):
                a0, a1, a2, a3, g0, g1 = accs
                off = i * (UNROLL * L)
                for s in range(UNROLL):
                    sl = buf[pl.ds(off + s * L, L)]
                    hit = jnp.where(off + s * L + iv == yspl, sl, 0.0)
                    if s % 2 == 0:
                        g0 = g0 + hit
                    else:
                        g1 = g1 + hit
                    if s % 4 == 0:
                        a0 = a0 + sl
                    elif s % 4 == 1:
                        a1 = a1 + sl
                    elif s % 4 == 2:
                        a2 = a2 + sl
                    else:
                        a3 = a3 + sl
                return a0, a1, a2, a3, g0, g1

            a0, a1, a2, a3, g0, g1 = lax.fori_loop(
                0, ITERS, reduce_body, (zero, zero, zero, zero, zero, zero))
            tot = (a0 + a1) + (a2 + a3)
            c0vec = jnp.where(iv == 0, buf[pl.ds(0, L)], 0.0)
            mf = jnp.where(yspl != PAD, fone, zero)
            gacc = gacc + mf * (g0 + g1)
            bacc = bacc + mf * (c0vec - tot)
            kacc = kacc + jnp.where(iv == 0, mf, zero)

        g_v[...] = gacc
        b_v[...] = bacc
        k_v[...] = kacc
        pltpu.sync_copy(g_v, g_hbm.at[wid])
        pltpu.sync_copy(b_v, b_hbm.at[wid])
        pltpu.sync_copy(k_v, k_hbm.at[wid])

    return k(x, y)


def _tc_body(y_ref, x_ref, b_ref, a_ref, k_ref):
    blk = x_ref[...]
    s = jnp.sum(blk, axis=1, keepdims=True)
    yv = y_ref[0]
    mask = yv != PAD
    col = lax.broadcasted_iota(jnp.int32, (BR, V), 1)
    g = jnp.sum(jnp.where((col == yv) & mask, blk, 0.0))
    b = jnp.sum(jnp.where(mask, blk[:, 0:1] - s, 0.0))
    cnt = jnp.sum(jnp.where(mask, 1.0, 0.0))
    r = pl.program_id(0)

    @pl.when(r == 0)
    def _():
        b_ref[0, 0] = b
        a_ref[0, 0] = g
        k_ref[0, 0] = cnt

    @pl.when(r != 0)
    def _():
        b_ref[0, 0] = b_ref[0, 0] + b
        a_ref[0, 0] = a_ref[0, 0] + g
        k_ref[0, 0] = k_ref[0, 0] + cnt


def _tc_part(x, y3):
    return pl.pallas_call(
        _tc_body,
        grid=(NRB,),
        in_specs=[
            pl.BlockSpec((1, BR, 1), lambda r: (r, 0, 0)),
            pl.BlockSpec((BR, V), lambda r: (r, 0)),
        ],
        out_specs=[
            pl.BlockSpec(memory_space=pltpu.SMEM),
            pl.BlockSpec(memory_space=pltpu.SMEM),
            pl.BlockSpec(memory_space=pltpu.SMEM),
        ],
        out_shape=[
            jax.ShapeDtypeStruct((1, 1), jnp.float32),
            jax.ShapeDtypeStruct((1, 1), jnp.float32),
            jax.ShapeDtypeStruct((1, 1), jnp.float32),
        ],
    )(y3, x)


def kernel(x, y):
    g_sc, b_sc, k_sc = _sc_part(x, y)
    b_tc, a_tc, k_tc = _tc_part(x, y.reshape(N // BR, BR, 1))
    a = a_tc[0, 0] + jnp.sum(g_sc)
    b = b_tc[0, 0] + jnp.sum(b_sc)
    cnt = k_tc[0, 0] + jnp.sum(k_sc)
    return (cnt * jnp.float32(ROW_CONST)
            - jnp.float32(CONF - EPS) * a
            + jnp.float32(EPS) * b)


# speedup vs baseline: 2.8911x; 1.1506x over previous
import functools
import math

import jax
import jax.numpy as jnp
from jax import lax
from jax.experimental import pallas as pl
from jax.experimental.pallas import tpu as pltpu
from jax.experimental.pallas import tpu_sc as plsc

N = 4096
V = 32000
PAD = 0
SMOOTH = 0.1
CONF = 1.0 - SMOOTH
EPS = SMOOTH / (V - 2)
ROW_CONST = CONF * math.log(CONF) + (V - 2) * EPS * math.log(EPS)

NC = 2
NS = 16
L = 16
NW = NC * NS

ROWS_SC = 1024
ROWS_TC = N - ROWS_SC
RPW = ROWS_SC // NW
IDX_BATCH = 128
SPLAT_BATCHES = RPW * L // IDX_BATCH

UNROLL = 25
SLICES = V // L
ITERS = SLICES // UNROLL

BR = 128
NRB = ROWS_TC // BR


def _sc_part(x, y):
    mesh = plsc.VectorSubcoreMesh(core_axis_name="c", subcore_axis_name="s")

    @functools.partial(
        pl.kernel,
        mesh=mesh,
        out_type=[
            jax.ShapeDtypeStruct((NW, L), jnp.float32),
            jax.ShapeDtypeStruct((NW, L), jnp.float32),
            jax.ShapeDtypeStruct((NW, L), jnp.float32),
        ],
        scratch_types=[
            pltpu.VMEM((RPW * L,), jnp.int32),
            pltpu.VMEM((RPW * L,), jnp.int32),
            pltpu.VMEM((2, V), jnp.float32),
            pltpu.VMEM((L,), jnp.float32),
            pltpu.VMEM((L,), jnp.float32),
            pltpu.VMEM((L,), jnp.float32),
            pltpu.SemaphoreType.DMA,
            pltpu.SemaphoreType.DMA,
            pltpu.SemaphoreType.DMA,
        ],
    )
    def k(x_hbm, y_hbm, g_hbm, b_hbm, k_hbm,
          idxs_v, yspl_v, rowbuf, g_v, b_v, k_v,
          sem, sem0, sem1):
        wid = lax.axis_index("s") * NC + lax.axis_index("c")
        iv = lax.iota(jnp.int32, L)
        row0 = ROWS_TC + wid * RPW

        zi = jnp.zeros((L,), jnp.int32)
        for j in range(RPW):
            idxs_v[pl.ds(j * L, L)] = zi + (row0 + j)
        for j in range(SPLAT_BATCHES):
            pltpu.async_copy(
                y_hbm.at[idxs_v.at[pl.ds(j * IDX_BATCH, IDX_BATCH)]],
                yspl_v.at[pl.ds(j * IDX_BATCH, IDX_BATCH)], sem).wait()

        sems = (sem0, sem1)
        cps = [pltpu.async_copy(x_hbm.at[row0], rowbuf.at[0], sem0), None]

        gacc = jnp.zeros((L,), jnp.float32)
        bacc = jnp.zeros((L,), jnp.float32)
        kacc = jnp.zeros((L,), jnp.float32)
        zero = jnp.zeros((L,), jnp.float32)
        fone = jnp.ones((L,), jnp.float32)

        for r in range(RPW):
            if r + 1 < RPW:
                cps[(r + 1) % 2] = pltpu.async_copy(
                    x_hbm.at[row0 + r + 1], rowbuf.at[(r + 1) % 2],
                    sems[(r + 1) % 2])
            cps[r % 2].wait()
            buf = rowbuf.at[r % 2]
            yspl = yspl_v[pl.ds(r * L, L)]

            @plsc.parallel_loop(0, V, step=UNROLL * L, unroll=2,
                                carry=(zero, zero, zero, zero, zero, zero))
            def reduce_body(off, accs):
                a0, a1, a2, a3, g0, g1 = accs
                for s in range(UNROLL):
                    sl = buf[pl.ds(off + s * L, L)]
                    hit = jnp.where(off + s * L + iv == yspl, sl, 0.0)
                    if s % 2 == 0:
                        g0 = g0 + hit
                    else:
                        g1 = g1 + hit
                    if s % 4 == 0:
                        a0 = a0 + sl
                    elif s % 4 == 1:
                        a1 = a1 + sl
                    elif s % 4 == 2:
                        a2 = a2 + sl
                    else:
                        a3 = a3 + sl
                return a0, a1, a2, a3, g0, g1

            a0, a1, a2, a3, g0, g1 = reduce_body
            tot = (a0 + a1) + (a2 + a3)
            c0vec = jnp.where(iv == 0, buf[pl.ds(0, L)], 0.0)
            mf = jnp.where(yspl != PAD, fone, zero)
            gacc = gacc + mf * (g0 + g1)
            bacc = bacc + mf * (c0vec - tot)
            kacc = kacc + jnp.where(iv == 0, mf, zero)

        g_v[...] = gacc
        b_v[...] = bacc
        k_v[...] = kacc
        pltpu.sync_copy(g_v, g_hbm.at[wid])
        pltpu.sync_copy(b_v, b_hbm.at[wid])
        pltpu.sync_copy(k_v, k_hbm.at[wid])

    return k(x, y)


def _tc_body(y_ref, x_ref, p_ref, k_ref):
    blk = x_ref[...]
    yv = y_ref[0]
    mask = yv != PAD
    col = lax.broadcasted_iota(jnp.int32, (BR, V), 1)
    w = jnp.where(col == yv, jnp.float32(CONF), jnp.float32(EPS))
    prow = jnp.sum(w * blk, axis=1, keepdims=True) - jnp.float32(EPS) * blk[:, 0:1]
    p = jnp.sum(jnp.where(mask, prow, 0.0))
    cnt = jnp.sum(jnp.where(mask, 1.0, 0.0))
    r = pl.program_id(0)

    @pl.when(r == 0)
    def _():
        p_ref[0, 0] = p
        k_ref[0, 0] = cnt

    @pl.when(r != 0)
    def _():
        p_ref[0, 0] = p_ref[0, 0] + p
        k_ref[0, 0] = k_ref[0, 0] + cnt


def _tc_part(x, y3):
    return pl.pallas_call(
        _tc_body,
        grid=(NRB,),
        in_specs=[
            pl.BlockSpec((1, BR, 1), lambda r: (r, 0, 0)),
            pl.BlockSpec((BR, V), lambda r: (r, 0)),
        ],
        out_specs=[
            pl.BlockSpec(memory_space=pltpu.SMEM),
            pl.BlockSpec(memory_space=pltpu.SMEM),
        ],
        out_shape=[
            jax.ShapeDtypeStruct((1, 1), jnp.float32),
            jax.ShapeDtypeStruct((1, 1), jnp.float32),
        ],
    )(y3, x)


def kernel(x, y):
    g_sc, b_sc, k_sc = _sc_part(x, y)
    p_tc, k_tc = _tc_part(x, y.reshape(N // BR, BR, 1))
    p = (p_tc[0, 0]
         + jnp.float32(CONF - EPS) * jnp.sum(g_sc)
         - jnp.float32(EPS) * jnp.sum(b_sc))
    cnt = k_tc[0, 0] + jnp.sum(k_sc)
    return cnt * jnp.float32(ROW_CONST) - p
